# Initial kernel scaffold; baseline (speedup 1.0000x reference)
import functools

import jax
import jax.numpy as jnp
from jax.experimental import pallas as pl
from jax.experimental.pallas import tpu as pltpu

_THRESHOLD = 0.5


def _tile_kernel(logit_ref, center_ref, out3_ref, cls_ref, mask_ref, *, hb, w, c):
    x = logit_ref[0]
    m = x[:, :, 0]
    cls = jnp.zeros(m.shape, dtype=jnp.int32)
    for k in range(1, c):
        xk = x[:, :, k]
        gt = xk > m
        m = jnp.where(gt, xk, m)
        cls = jnp.where(gt, k, cls)
    s = jnp.zeros(m.shape, dtype=jnp.float32)
    for k in range(c):
        s = s + jnp.exp(x[:, :, k] - m)
    score = 1.0 / s
    mask = score > _THRESHOLD
    mf = mask.astype(jnp.float32)

    row0 = pl.program_id(1) * hb
    yy = jax.lax.broadcasted_iota(jnp.float32, (hb, w), 0) + row0
    xx = jax.lax.broadcasted_iota(jnp.float32, (hb, w), 1)
    sy = (yy + center_ref[0, :, :, 0]) * mf
    sx = (xx + center_ref[0, :, :, 1]) * mf

    out3_ref[0] = jnp.stack([jnp.where(mask, score, 0.0), sy, sx], axis=-1)
    cls_ref[0] = cls
    mask_ref[0] = mask


def kernel(segm_logit, center_point):
    B, H, W, C = segm_logit.shape
    HB = 128
    grid = (B, H // HB)
    out3, cls, mask = pl.pallas_call(
        functools.partial(_tile_kernel, hb=HB, w=W, c=C),
        grid=grid,
        in_specs=[
            pl.BlockSpec((1, HB, W, C), lambda b, h: (b, h, 0, 0)),
            pl.BlockSpec((1, HB, W, 2), lambda b, h: (b, h, 0, 0)),
        ],
        out_specs=[
            pl.BlockSpec((1, HB, W, 3), lambda b, h: (b, h, 0, 0)),
            pl.BlockSpec((1, HB, W), lambda b, h: (b, h, 0)),
            pl.BlockSpec((1, HB, W), lambda b, h: (b, h, 0)),
        ],
        out_shape=[
            jax.ShapeDtypeStruct((B, H, W, 3), jnp.float32),
            jax.ShapeDtypeStruct((B, H, W), jnp.int32),
            jax.ShapeDtypeStruct((B, H, W), jnp.bool_),
        ],
        compiler_params=pltpu.CompilerParams(
            dimension_semantics=("parallel", "parallel"),
        ),
    )(segm_logit, center_point)
    return (out3, cls.astype(jnp.int64), mask)


# speedup vs baseline: 1.6472x; 1.6472x over previous
import functools

import jax
import jax.numpy as jnp
from jax.experimental import pallas as pl
from jax.experimental.pallas import tpu as pltpu

_THRESHOLD = 0.5


def _tile_kernel(logit_ref, center_ref, out3_ref, cls_ref, mask_ref, *, hb, w, c, h):
    m = logit_ref[0]
    cls = jnp.zeros(m.shape, dtype=jnp.int32)
    for k in range(1, c):
        xk = logit_ref[k]
        gt = xk > m
        m = jnp.where(gt, xk, m)
        cls = jnp.where(gt, k, cls)
    s = jnp.zeros(m.shape, dtype=jnp.float32)
    for k in range(c):
        s = s + jnp.exp(logit_ref[k] - m)
    score = 1.0 / s
    mask = score > _THRESHOLD
    mf = mask.astype(jnp.float32)

    row0 = (pl.program_id(0) * hb) % h
    yy = (jax.lax.broadcasted_iota(jnp.int32, (hb, w), 0) + row0).astype(jnp.float32)
    xx = jax.lax.broadcasted_iota(jnp.int32, (hb, w), 1).astype(jnp.float32)

    out3_ref[0] = jnp.where(mask, score, 0.0)
    out3_ref[1] = (yy + center_ref[0]) * mf
    out3_ref[2] = (xx + center_ref[1]) * mf
    cls_ref[...] = cls
    mask_ref[...] = mask


def kernel(segm_logit, center_point):
    B, H, W, C = segm_logit.shape
    HB = 128
    grid = (B * H // HB,)
    logit_t = jnp.transpose(segm_logit.reshape(B * H, W, C), (2, 0, 1))
    center_t = jnp.transpose(center_point.reshape(B * H, W, 2), (2, 0, 1))
    out3_t, cls, mask = pl.pallas_call(
        functools.partial(_tile_kernel, hb=HB, w=W, c=C, h=H),
        grid=grid,
        in_specs=[
            pl.BlockSpec((C, HB, W), lambda i: (0, i, 0)),
            pl.BlockSpec((2, HB, W), lambda i: (0, i, 0)),
        ],
        out_specs=[
            pl.BlockSpec((3, HB, W), lambda i: (0, i, 0)),
            pl.BlockSpec((HB, W), lambda i: (i, 0)),
            pl.BlockSpec((HB, W), lambda i: (i, 0)),
        ],
        out_shape=[
            jax.ShapeDtypeStruct((3, B * H, W), jnp.float32),
            jax.ShapeDtypeStruct((B * H, W), jnp.int32),
            jax.ShapeDtypeStruct((B * H, W), jnp.bool_),
        ],
        compiler_params=pltpu.CompilerParams(
            dimension_semantics=("arbitrary",),
        ),
    )(logit_t, center_t)
    return (
        jnp.transpose(out3_t, (1, 2, 0)).reshape(B, H, W, 3),
        cls.reshape(B, H, W).astype(jnp.int64),
        mask.reshape(B, H, W),
    )
